# baseline (speedup 1.0000x reference)
import jax
import jax.numpy as jnp
from jax.experimental import pallas as pl
from jax.experimental.pallas import tpu as pltpu


def _mlp_t_kernel(x_ref, xh_ref, w1x_ref, w1h_ref, w2_ref, w3_ref, w4_ref,
                  b1_ref, b2_ref, b3_ref, b4_ref, o_ref):
    def sig(v):
        return 0.5 * jnp.tanh(0.5 * v) + 0.5

    def chain(xc, xhc):
        h = jnp.dot(w1x_ref[...], xc, preferred_element_type=jnp.float32)
        h = h + jnp.dot(w1h_ref[...], xhc, preferred_element_type=jnp.float32)
        h = sig(h + b1_ref[...])
        h = sig(jnp.dot(w2_ref[...], h, preferred_element_type=jnp.float32)
                + b2_ref[...])
        h = sig(jnp.dot(w3_ref[...], h, preferred_element_type=jnp.float32)
                + b3_ref[...])
        h = jnp.dot(w4_ref[...], h, preferred_element_type=jnp.float32)
        return sig(h[0:1, :] + b4_ref[0:1, :])

    c = x_ref.shape[1] // 2
    o_ref[:, :c] = chain(x_ref[:, :c], xh_ref[:, :c]).astype(o_ref.dtype)
    o_ref[:, c:] = chain(x_ref[:, c:], xh_ref[:, c:]).astype(o_ref.dtype)


def kernel(x, x_hat, w1, b1, w2, b2, w3, b3, w4, b4, *, batch_tile=8192):
    bs = x.shape[0]
    feat = x.shape[1] * x.shape[2]

    xf = x.transpose(1, 2, 0).reshape(feat, bs).astype(jnp.float32)
    xhf = x_hat.transpose(1, 2, 0).reshape(feat, bs).astype(jnp.float32)

    w1f = w1.astype(jnp.float32)
    w1x = w1f[:feat].T
    w1h = w1f[feat:].T
    w2t = w2.astype(jnp.float32).T
    w3t = w3.astype(jnp.float32).T
    w4t = jnp.zeros((8, 16), jnp.float32).at[0:1, :].set(
        w4.astype(jnp.float32).T)
    b1c = b1.astype(jnp.float32).reshape(-1, 1)
    b2c = b2.astype(jnp.float32).reshape(-1, 1)
    b3c = b3.astype(jnp.float32).reshape(-1, 1)
    b4c = jnp.zeros((8, 1), jnp.float32).at[0:1, :].set(
        b4.astype(jnp.float32).reshape(1, 1))

    bt = min(batch_tile, bs)
    pad = (-bs) % bt
    if pad:
        xf = jnp.pad(xf, ((0, 0), (0, pad)))
        xhf = jnp.pad(xhf, ((0, 0), (0, pad)))
    bs_p = bs + pad
    grid = bs_p // bt

    out = pl.pallas_call(
        _mlp_t_kernel,
        out_shape=jax.ShapeDtypeStruct((1, bs_p), jnp.float32),
        grid=(grid,),
        in_specs=[
            pl.BlockSpec((feat, bt), lambda i: (0, i)),
            pl.BlockSpec((feat, bt), lambda i: (0, i)),
            pl.BlockSpec(w1x.shape, lambda i: (0, 0)),
            pl.BlockSpec(w1h.shape, lambda i: (0, 0)),
            pl.BlockSpec(w2t.shape, lambda i: (0, 0)),
            pl.BlockSpec(w3t.shape, lambda i: (0, 0)),
            pl.BlockSpec(w4t.shape, lambda i: (0, 0)),
            pl.BlockSpec(b1c.shape, lambda i: (0, 0)),
            pl.BlockSpec(b2c.shape, lambda i: (0, 0)),
            pl.BlockSpec(b3c.shape, lambda i: (0, 0)),
            pl.BlockSpec(b4c.shape, lambda i: (0, 0)),
        ],
        out_specs=pl.BlockSpec((1, bt), lambda i: (0, i)),
        compiler_params=pltpu.CompilerParams(
            dimension_semantics=("parallel",)),
    )(xf, xhf, w1x, w1h, w2t, w3t, w4t, b1c, b2c, b3c, b4c)

    return out[0, :bs].reshape(bs, 1)


# speedup vs baseline: 1.0956x; 1.0956x over previous
import jax
import jax.numpy as jnp
from jax.experimental import pallas as pl
from jax.experimental.pallas import tpu as pltpu


def _mlp_t_kernel(x_ref, xh_ref, w1_ref, w2_ref, w3_ref, w4_ref,
                  b1_ref, b2_ref, b3_ref, b4_ref, o_ref):
    def sig(v):
        return 0.5 * jnp.tanh(0.5 * v) + 0.5

    xall = jnp.concatenate([x_ref[...], xh_ref[...]], axis=0)
    h = jnp.dot(w1_ref[...], xall, preferred_element_type=jnp.float32)
    h = sig(h + b1_ref[...])
    h = sig(
        jnp.dot(w2_ref[...], h, preferred_element_type=jnp.float32)
        + b2_ref[...])
    h = sig(
        jnp.dot(w3_ref[...], h, preferred_element_type=jnp.float32)
        + b3_ref[...])
    h = jnp.dot(w4_ref[...], h, preferred_element_type=jnp.float32)
    h = sig(h[0:1, :] + b4_ref[0:1, :])
    o_ref[...] = h.astype(o_ref.dtype)


def kernel(x, x_hat, w1, b1, w2, b2, w3, b3, w4, b4, *, batch_tile=8192):
    bs = x.shape[0]
    feat = x.shape[1] * x.shape[2]

    xf = x.transpose(1, 2, 0).reshape(feat, bs).astype(jnp.float32)
    xhf = x_hat.transpose(1, 2, 0).reshape(feat, bs).astype(jnp.float32)

    w1t = w1.astype(jnp.float32).T
    w2t = w2.astype(jnp.float32).T
    w3t = w3.astype(jnp.float32).T
    w4t = jnp.zeros((8, 16), jnp.float32).at[0:1, :].set(
        w4.astype(jnp.float32).T)
    b1c = b1.astype(jnp.float32).reshape(-1, 1)
    b2c = b2.astype(jnp.float32).reshape(-1, 1)
    b3c = b3.astype(jnp.float32).reshape(-1, 1)
    b4c = jnp.zeros((8, 1), jnp.float32).at[0:1, :].set(
        b4.astype(jnp.float32).reshape(1, 1))

    bt = min(batch_tile, bs)
    pad = (-bs) % bt
    if pad:
        xf = jnp.pad(xf, ((0, 0), (0, pad)))
        xhf = jnp.pad(xhf, ((0, 0), (0, pad)))
    bs_p = bs + pad
    grid = bs_p // bt

    out = pl.pallas_call(
        _mlp_t_kernel,
        out_shape=jax.ShapeDtypeStruct((1, bs_p), jnp.float32),
        grid=(grid,),
        in_specs=[
            pl.BlockSpec((feat, bt), lambda i: (0, i)),
            pl.BlockSpec((feat, bt), lambda i: (0, i)),
            pl.BlockSpec(w1t.shape, lambda i: (0, 0)),
            pl.BlockSpec(w2t.shape, lambda i: (0, 0)),
            pl.BlockSpec(w3t.shape, lambda i: (0, 0)),
            pl.BlockSpec(w4t.shape, lambda i: (0, 0)),
            pl.BlockSpec(b1c.shape, lambda i: (0, 0)),
            pl.BlockSpec(b2c.shape, lambda i: (0, 0)),
            pl.BlockSpec(b3c.shape, lambda i: (0, 0)),
            pl.BlockSpec(b4c.shape, lambda i: (0, 0)),
        ],
        out_specs=pl.BlockSpec((1, bt), lambda i: (0, i)),
        compiler_params=pltpu.CompilerParams(
            dimension_semantics=("parallel",)),
    )(xf, xhf, w1t, w2t, w3t, w4t, b1c, b2c, b3c, b4c)

    return out[0, :bs].reshape(bs, 1)


# speedup vs baseline: 1.1321x; 1.0334x over previous
import jax
import jax.numpy as jnp
from jax.experimental import pallas as pl
from jax.experimental.pallas import tpu as pltpu


def _mlp_t_kernel(x_ref, xh_ref, w1_ref, w2_ref, w3_ref, w4_ref,
                  b1_ref, b2_ref, b3_ref, b4_ref, o_ref):
    xall = jnp.concatenate([x_ref[...], xh_ref[...]], axis=0)
    t = jnp.tanh(jnp.dot(w1_ref[...], xall,
                         preferred_element_type=jnp.float32) + b1_ref[...])
    t = jnp.tanh(jnp.dot(w2_ref[...], t,
                         preferred_element_type=jnp.float32) + b2_ref[...])
    t = jnp.tanh(jnp.dot(w3_ref[...], t,
                         preferred_element_type=jnp.float32) + b3_ref[...])
    t = jnp.dot(w4_ref[...], t, preferred_element_type=jnp.float32)
    y = 0.5 * jnp.tanh(t[0:1, :] + b4_ref[0:1, :]) + 0.5
    o_ref[...] = y.astype(o_ref.dtype)


def kernel(x, x_hat, w1, b1, w2, b2, w3, b3, w4, b4, *, batch_tile=8192):
    bs = x.shape[0]
    feat = x.shape[1] * x.shape[2]

    xf = x.transpose(1, 2, 0).reshape(feat, bs).astype(jnp.float32)
    xhf = x_hat.transpose(1, 2, 0).reshape(feat, bs).astype(jnp.float32)

    w1t = 0.5 * w1.astype(jnp.float32).T
    b1c = 0.5 * b1.astype(jnp.float32).reshape(-1, 1)

    def fold(w, b):
        wt = w.astype(jnp.float32).T
        bc = b.astype(jnp.float32).reshape(-1, 1)
        return 0.25 * wt, 0.5 * (0.5 * wt.sum(axis=1, keepdims=True) + bc)

    w2t, b2c = fold(w2, b2)
    w3t, b3c = fold(w3, b3)
    w4f, b4f = fold(w4, b4)
    w4t = jnp.zeros((8, 16), jnp.float32).at[0:1, :].set(w4f)
    b4c = jnp.zeros((8, 1), jnp.float32).at[0:1, :].set(b4f)

    bt = min(batch_tile, bs)
    pad = (-bs) % bt
    if pad:
        xf = jnp.pad(xf, ((0, 0), (0, pad)))
        xhf = jnp.pad(xhf, ((0, 0), (0, pad)))
    bs_p = bs + pad
    grid = bs_p // bt

    out = pl.pallas_call(
        _mlp_t_kernel,
        out_shape=jax.ShapeDtypeStruct((1, bs_p), jnp.float32),
        grid=(grid,),
        in_specs=[
            pl.BlockSpec((feat, bt), lambda i: (0, i)),
            pl.BlockSpec((feat, bt), lambda i: (0, i)),
            pl.BlockSpec(w1t.shape, lambda i: (0, 0)),
            pl.BlockSpec(w2t.shape, lambda i: (0, 0)),
            pl.BlockSpec(w3t.shape, lambda i: (0, 0)),
            pl.BlockSpec(w4t.shape, lambda i: (0, 0)),
            pl.BlockSpec(b1c.shape, lambda i: (0, 0)),
            pl.BlockSpec(b2c.shape, lambda i: (0, 0)),
            pl.BlockSpec(b3c.shape, lambda i: (0, 0)),
            pl.BlockSpec(b4c.shape, lambda i: (0, 0)),
        ],
        out_specs=pl.BlockSpec((1, bt), lambda i: (0, i)),
        compiler_params=pltpu.CompilerParams(
            dimension_semantics=("parallel",)),
    )(xf, xhf, w1t, w2t, w3t, w4t, b1c, b2c, b3c, b4c)

    return out[0, :bs].reshape(bs, 1)


# speedup vs baseline: 1.1699x; 1.0334x over previous
import jax
import jax.numpy as jnp
from jax.experimental import pallas as pl
from jax.experimental.pallas import tpu as pltpu


def _mlp_t_kernel(x_ref, xh_ref, w1_ref, w2_ref, w3_ref, w4_ref,
                  b1_ref, b2_ref, b3_ref, b4_ref, o_ref):
    xall = jnp.concatenate([x_ref[...], xh_ref[...]], axis=0)
    t = jnp.tanh(jnp.dot(w1_ref[...], xall,
                         preferred_element_type=jnp.float32) + b1_ref[...])
    t = jnp.tanh(jnp.dot(w2_ref[...], t,
                         preferred_element_type=jnp.float32) + b2_ref[...])
    t = jnp.tanh(jnp.dot(w3_ref[...], t,
                         preferred_element_type=jnp.float32) + b3_ref[...])
    t = jnp.dot(w4_ref[...], t, preferred_element_type=jnp.float32)
    y = 0.5 * jnp.tanh(t[0:1, :] + b4_ref[0:1, :]) + 0.5
    o_ref[...] = y.astype(o_ref.dtype)


def kernel(x, x_hat, w1, b1, w2, b2, w3, b3, w4, b4, *, batch_tile=16384):
    bs = x.shape[0]
    feat = x.shape[1] * x.shape[2]

    xf = x.transpose(1, 2, 0).reshape(feat, bs).astype(jnp.float32)
    xhf = x_hat.transpose(1, 2, 0).reshape(feat, bs).astype(jnp.float32)

    w1t = 0.5 * w1.astype(jnp.float32).T
    b1c = 0.5 * b1.astype(jnp.float32).reshape(-1, 1)

    def fold(w, b):
        wt = w.astype(jnp.float32).T
        bc = b.astype(jnp.float32).reshape(-1, 1)
        return 0.25 * wt, 0.5 * (0.5 * wt.sum(axis=1, keepdims=True) + bc)

    w2t, b2c = fold(w2, b2)
    w3t, b3c = fold(w3, b3)
    w4f, b4f = fold(w4, b4)
    w4t = jnp.zeros((8, 16), jnp.float32).at[0:1, :].set(w4f)
    b4c = jnp.zeros((8, 1), jnp.float32).at[0:1, :].set(b4f)

    bt = min(batch_tile, bs)
    pad = (-bs) % bt
    if pad:
        xf = jnp.pad(xf, ((0, 0), (0, pad)))
        xhf = jnp.pad(xhf, ((0, 0), (0, pad)))
    bs_p = bs + pad
    grid = bs_p // bt

    out = pl.pallas_call(
        _mlp_t_kernel,
        out_shape=jax.ShapeDtypeStruct((1, bs_p), jnp.float32),
        grid=(grid,),
        in_specs=[
            pl.BlockSpec((feat, bt), lambda i: (0, i)),
            pl.BlockSpec((feat, bt), lambda i: (0, i)),
            pl.BlockSpec(w1t.shape, lambda i: (0, 0)),
            pl.BlockSpec(w2t.shape, lambda i: (0, 0)),
            pl.BlockSpec(w3t.shape, lambda i: (0, 0)),
            pl.BlockSpec(w4t.shape, lambda i: (0, 0)),
            pl.BlockSpec(b1c.shape, lambda i: (0, 0)),
            pl.BlockSpec(b2c.shape, lambda i: (0, 0)),
            pl.BlockSpec(b3c.shape, lambda i: (0, 0)),
            pl.BlockSpec(b4c.shape, lambda i: (0, 0)),
        ],
        out_specs=pl.BlockSpec((1, bt), lambda i: (0, i)),
        compiler_params=pltpu.CompilerParams(
            dimension_semantics=("parallel",)),
    )(xf, xhf, w1t, w2t, w3t, w4t, b1c, b2c, b3c, b4c)

    return out[0, :bs].reshape(bs, 1)
